# baseline (speedup 1.0000x reference)
import functools

import jax
import jax.numpy as jnp
from jax import lax
from jax.experimental import pallas as pl
from jax.experimental.pallas import tpu as pltpu
from jax.experimental.pallas import tpu_sc as plsc

N = 100000
E = 3200000
NUM_LOCATIONS = 15
HEADS = 4
CPH = 3
HID = 12

NP = 100352
BLK = 2048
GRID = NP // BLK

NC, NS = 2, 16
NW = NC * NS
EROW = 128
ERP = 25088
EP = ERP * EROW
WR = ERP // NW
SR = 56
NSC = WR // SR
RPS = NP // NS


def _prep_body(tid_ref, req_ref, upd_ref, tw_ref, wr_ref, wu_ref,
               st_ref, ald_ref, amax_ref):
    i = pl.program_id(0)
    tid = tid_ref[...]
    tw = tw_ref[...]
    r = req_ref[...] * wr_ref[...] + upd_ref[...] * wu_ref[...]
    for k in range(4):
        r = r + jnp.where(tid == k, 1.0, 0.0) * tw[k:k + 1, :]
    st_ref[...] = r[:, :16]
    ald_ref[...] = jnp.concatenate(
        [r[:, 16:20], jnp.zeros((BLK, 12), jnp.float32)], axis=1)
    bm = jnp.max(r[:, :4], axis=0)
    bm16 = jnp.concatenate([bm, jnp.full((12,), -1e30, jnp.float32)])[None, :]
    prev = jnp.where(i == 0, jnp.full((1, 16), -1e30, jnp.float32),
                     amax_ref[...])
    amax_ref[...] = jnp.maximum(prev, bm16)


def _epilogue(acc0_ref, acc1_ref, st_ref, ald_ref, amax_ref, b_ref):
    st = st_ref[...]
    als = st[:, :4]
    h = st[:, 4:16]
    ald = ald_ref[...][:, :4]
    am = amax_ref[...][:, :4]
    z = als + ald
    lr = jnp.maximum(z, 0.2 * z)
    q = am + ald
    m = jnp.maximum(q, 0.2 * q)
    ee = jnp.exp(lr - m)
    a0 = acc0_ref[...]
    a1 = acc1_ref[...]
    den = a0[:, :4] + a1[:, :4] + ee
    ee3 = jnp.concatenate(
        [ee[:, hh:hh + 1] for hh in range(HEADS) for _ in range(CPH)], axis=1)
    den3 = jnp.concatenate(
        [den[:, hh:hh + 1] for hh in range(HEADS) for _ in range(CPH)], axis=1)
    num = a0[:, 4:] + a1[:, 4:] + ee3 * h
    return num / (den3 + 1e-16) + b_ref[...]


def _epi_prep_body(acc0_ref, acc1_ref, st_ref, ald_ref, amax_ref, b_ref,
                   wcat_ref, st2_ref, ald2_ref, amax2_ref):
    i = pl.program_id(0)
    out = _epilogue(acc0_ref, acc1_ref, st_ref, ald_ref, amax_ref, b_ref)
    x2 = jnp.maximum(out, 0.0)
    r = jnp.dot(x2, wcat_ref[...], preferred_element_type=jnp.float32)
    st2_ref[...] = r[:, :16]
    ald2_ref[...] = jnp.concatenate(
        [r[:, 16:20], jnp.zeros((BLK, 12), jnp.float32)], axis=1)
    bm = jnp.max(r[:, :4], axis=0)
    bm16 = jnp.concatenate([bm, jnp.full((12,), -1e30, jnp.float32)])[None, :]
    prev = jnp.where(i == 0, jnp.full((1, 16), -1e30, jnp.float32),
                     amax2_ref[...])
    amax2_ref[...] = jnp.maximum(prev, bm16)


def _final_body(acc0_ref, acc1_ref, st_ref, ald_ref, amax_ref, b_ref,
                w0_ref, b0_ref, w1_ref, b1_ref, w2_ref, b2_ref,
                w3_ref, b3_ref, out_ref):
    i = pl.program_id(0)
    out = _epilogue(acc0_ref, acc1_ref, st_ref, ald_ref, amax_ref, b_ref)
    v = jnp.maximum(jnp.dot(out, w0_ref[...],
                            preferred_element_type=jnp.float32) + b0_ref[...],
                    0.0)
    v = jnp.maximum(jnp.dot(v, w1_ref[...],
                            preferred_element_type=jnp.float32) + b1_ref[...],
                    0.0)
    v = jnp.maximum(jnp.dot(v, w2_ref[...],
                            preferred_element_type=jnp.float32) + b2_ref[...],
                    0.0)
    nv = jnp.dot(v, w3_ref[...],
                 preferred_element_type=jnp.float32) + b3_ref[...]
    rowid = i * BLK + lax.broadcasted_iota(jnp.int32, (BLK, 1), 0)
    nv = jnp.where(rowid < N, nv, 0.0)
    s = (jnp.sum(nv) * jnp.float32(1.0 / N)).reshape(1, 1)
    out_ref[...] = jnp.where(i == 0, s, out_ref[...] + s)


def _node_spec(width):
    return pl.BlockSpec((BLK, width), lambda i: (i, 0))


def _const_spec(shape):
    nd = len(shape)
    return pl.BlockSpec(shape, lambda i: (0,) * nd)


def _prep(tidf, reqp, updp, tw, wr, wu):
    return pl.pallas_call(
        _prep_body,
        grid=(GRID,),
        in_specs=[_node_spec(1), _node_spec(1), _node_spec(1),
                  _const_spec((4, 20)), _const_spec((1, 20)),
                  _const_spec((1, 20))],
        out_specs=[_node_spec(16), _node_spec(16), _const_spec((1, 16))],
        out_shape=[
            jax.ShapeDtypeStruct((NP, 16), jnp.float32),
            jax.ShapeDtypeStruct((NP, 16), jnp.float32),
            jax.ShapeDtypeStruct((1, 16), jnp.float32),
        ],
    )(tidf, reqp, updp, tw, wr, wu)


def _epi_prep(acc, st, ald, amax, b2d, wcat):
    return pl.pallas_call(
        _epi_prep_body,
        grid=(GRID,),
        in_specs=[_node_spec(16), _node_spec(16), _node_spec(16),
                  _node_spec(16), _const_spec((1, 16)), _const_spec((1, 12)),
                  _const_spec((12, 20))],
        out_specs=[_node_spec(16), _node_spec(16), _const_spec((1, 16))],
        out_shape=[
            jax.ShapeDtypeStruct((NP, 16), jnp.float32),
            jax.ShapeDtypeStruct((NP, 16), jnp.float32),
            jax.ShapeDtypeStruct((1, 16), jnp.float32),
        ],
    )(acc[0], acc[1], st, ald, amax, b2d, wcat)


def _final(acc, st, ald, amax, b2d, cW0, cb0, cW1, cb1, cW2, cb2, cW3, cb3):
    return pl.pallas_call(
        _final_body,
        grid=(GRID,),
        in_specs=[_node_spec(16), _node_spec(16), _node_spec(16),
                  _node_spec(16), _const_spec((1, 16)), _const_spec((1, 12)),
                  _const_spec((12, 12)), _const_spec((1, 12)),
                  _const_spec((12, 12)), _const_spec((1, 12)),
                  _const_spec((12, 12)), _const_spec((1, 12)),
                  _const_spec((12, 1)), _const_spec((1, 1))],
        out_specs=_const_spec((1, 1)),
        out_shape=jax.ShapeDtypeStruct((1, 1), jnp.float32),
    )(acc[0], acc[1], st, ald, amax, b2d,
      cW0, cb0.reshape(1, 12), cW1, cb1.reshape(1, 12),
      cW2, cb2.reshape(1, 12), cW3, cb3.reshape(1, 1))


_SC_MESH = plsc.VectorSubcoreMesh(core_axis_name="c", subcore_axis_name="s")


@functools.partial(
    pl.kernel,
    mesh=_SC_MESH,
    compiler_params=pltpu.CompilerParams(needs_layout_passes=False,
                                         use_tc_tiling_on_sc=False),
    out_type=jax.ShapeDtypeStruct((NC, NP, 16), jnp.float32),
    scratch_types=[
        pltpu.VMEM((SR, EROW), jnp.int32),
        pltpu.VMEM((SR, EROW), jnp.int32),
        pltpu.VMEM((EROW, 16), jnp.float32),
        pltpu.VMEM((EROW, 16), jnp.float32),
        pltpu.VMEM((EROW, 16), jnp.float32),
        pltpu.VMEM((EROW, 16), jnp.float32),
        pltpu.VMEM((EROW, 16), jnp.float32),
        pltpu.VMEM((EROW, 16), jnp.float32),
        pltpu.VMEM((1, 16), jnp.float32),
        pltpu.VMEM_SHARED((NP, 16), jnp.float32),
        pltpu.SemaphoreType.DMA,
        pltpu.SemaphoreType.DMA,
        pltpu.SemaphoreType.DMA,
        pltpu.SemaphoreType.DMA,
    ],
)
def _sc_edge(st_hbm, ald_hbm, amax_hbm, src_hbm, dst_hbm, out_hbm,
             sidx, didx, srows0, srows1, aldrows0, aldrows1, vals0, vals1,
             amaxv, acc, gsem0, gsem1, ssem0, ssem1):
    c = lax.axis_index("c")
    s = lax.axis_index("s")
    wid = c * NS + s

    pltpu.sync_copy(amax_hbm, amaxv)
    base = s * RPS

    zero16 = jnp.zeros((16,), jnp.float32)

    def zfill(i, _):
        vals0[i, :] = zero16
        return 0

    lax.fori_loop(0, EROW, zfill, 0)

    def zcopy(j, _):
        pltpu.sync_copy(vals0, acc.at[pl.ds(base + j * EROW, EROW)])
        return 0

    lax.fori_loop(0, RPS // EROW, zcopy, 0)

    plsc.subcore_barrier()

    lanes0 = lax.iota(jnp.int32, 16)
    zl = jnp.zeros((16,), jnp.int32)
    am = [plsc.load_gather(amaxv, [zl, jnp.full((16,), hh, jnp.int32)])
          for hh in range(HEADS)]

    row0 = wid * WR

    srows = (srows0, srows1)
    aldrows = (aldrows0, aldrows1)
    vals = (vals0, vals1)
    gsem = (gsem0, gsem1)
    ssem = (ssem0, ssem1)

    def chunk_body(t, _):
        r0 = row0 + t * SR
        pltpu.sync_copy(src_hbm.at[pl.ds(r0, SR)], sidx)
        pltpu.sync_copy(dst_hbm.at[pl.ds(r0, SR)], didx)

        pltpu.async_copy(st_hbm.at[sidx.at[0]], srows0, gsem0)
        pltpu.async_copy(ald_hbm.at[didx.at[0]], aldrows0, gsem0)

        def pair_body(g, _):
            for b in range(2):
                r = g * 2 + b
                nb = 1 - b

                @pl.when(r + 1 < SR)
                def _start_next():
                    pltpu.async_copy(st_hbm.at[sidx.at[r + 1]],
                                     srows[nb], gsem[nb])
                    pltpu.async_copy(ald_hbm.at[didx.at[r + 1]],
                                     aldrows[nb], gsem[nb])

                pltpu.make_async_copy(st_hbm.at[sidx.at[r]],
                                      srows[b], gsem[b]).wait()
                pltpu.make_async_copy(ald_hbm.at[didx.at[r]],
                                      aldrows[b], gsem[b]).wait()

                @pl.when(r >= 2)
                def _wait_prev_scatter():
                    pltpu.make_async_copy(vals[b], acc.at[didx.at[r - 2]],
                                          ssem[b]).wait()

                def grp(gi, _, _b=b):
                    lanes = gi * 16 + lanes0
                    for hh in range(HEADS):
                        colh = jnp.full((16,), hh, jnp.int32)
                        als = plsc.load_gather(srows[_b], [lanes, colh])
                        ald = plsc.load_gather(aldrows[_b], [lanes, colh])
                        z = als + ald
                        lr = jnp.maximum(z, 0.2 * z)
                        q = am[hh] + ald
                        m = jnp.maximum(q, 0.2 * q)
                        ee = jnp.exp(lr - m)
                        plsc.store_scatter(vals[_b], [lanes, colh], ee)
                        for cc in range(CPH):
                            col = jnp.full((16,), 4 + hh * CPH + cc,
                                           jnp.int32)
                            hv = plsc.load_gather(srows[_b], [lanes, col])
                            plsc.store_scatter(vals[_b], [lanes, col],
                                               ee * hv)
                    return 0

                lax.fori_loop(0, EROW // 16, grp, 0)
                pltpu.async_copy(vals[b], acc.at[didx.at[r]], ssem[b],
                                 add=True)
            return 0

        lax.fori_loop(0, SR // 2, pair_body, 0)

        pltpu.make_async_copy(vals0, acc.at[didx.at[SR - 2]], ssem0).wait()
        pltpu.make_async_copy(vals1, acc.at[didx.at[SR - 1]], ssem1).wait()
        return 0

    lax.fori_loop(0, NSC, chunk_body, 0)

    plsc.subcore_barrier()

    pltpu.sync_copy(acc.at[pl.ds(base, RPS)],
                    out_hbm.at[c, pl.ds(base, RPS)])


def _wcat(W, a_src, a_dst):
    eye = jnp.eye(HEADS, dtype=jnp.float32)
    A_src = (eye[:, None, :] * a_src[:, :, None]).reshape(HID, HEADS)
    A_dst = (eye[:, None, :] * a_dst[:, :, None]).reshape(HID, HEADS)
    return jnp.concatenate([W @ A_src, W, W @ A_dst], axis=1)


def kernel(type_ids, update_step, requests, edge_index, latency, batch,
           type_emb, W0, a_src0, a_dst0, b0, W1, a_src1, a_dst1, b1,
           W2, a_src2, a_dst2, b2, W3, a_src3, a_dst3, b3,
           cW0, cb0, cW1, cb1, cW2, cb2, cW3, cb3):
    tail = requests[NUM_LOCATIONS:]
    mean_r = jnp.mean(tail)
    std_r = jnp.std(tail, ddof=1)
    scale = 1.0 / (std_r + 1e-06)
    reqf = jnp.where(jnp.arange(N) < NUM_LOCATIONS, requests,
                     (requests - mean_r) * scale)
    tidf = jnp.zeros((NP, 1), jnp.float32).at[:N, 0].set(
        type_ids.astype(jnp.float32))
    reqp = jnp.zeros((NP, 1), jnp.float32).at[:N, 0].set(reqf)
    updp = jnp.zeros((NP, 1), jnp.float32).at[:N, 0].set(update_step)

    npad = EP - E
    pad_idx = (N + (jnp.arange(npad, dtype=jnp.int32) % (NP - N))).astype(
        edge_index.dtype)
    src2d = jnp.concatenate([edge_index[0], pad_idx]).reshape(ERP, EROW)
    dst2d = jnp.concatenate([edge_index[1], pad_idx]).reshape(ERP, EROW)
    src2d = src2d.astype(jnp.int32)
    dst2d = dst2d.astype(jnp.int32)

    layers = [(W0, a_src0, a_dst0, b0), (W1, a_src1, a_dst1, b1),
              (W2, a_src2, a_dst2, b2), (W3, a_src3, a_dst3, b3)]

    wc0 = _wcat(*layers[0][:3])
    st, ald, amax = _prep(tidf, reqp, updp, type_emb @ wc0[:3],
                          wc0[3:4], wc0[4:5])
    for li in range(4):
        W, a_src, a_dst, b = layers[li]
        acc = _sc_edge(st, ald, amax, src2d, dst2d)
        if li < 3:
            Wn, a_srcn, a_dstn, _ = layers[li + 1]
            st, ald, amax = _epi_prep(acc, st, ald, amax, b.reshape(1, 12),
                                      _wcat(Wn, a_srcn, a_dstn))
        else:
            total = _final(acc, st, ald, amax, b.reshape(1, 12),
                           cW0, cb0, cW1, cb1, cW2, cb2, cW3, cb3)
    return total
